# Initial kernel scaffold; baseline (speedup 1.0000x reference)
import jax
import jax.numpy as jnp
from jax.experimental import pallas as pl


def kernel(feat, edge_index, edge_weight, W, b_fc, bias, coef_self, coef_posi, coef_nega):
    raise NotImplementedError("write your pallas kernel here")


# speedup vs baseline: 18.8555x; 18.8555x over previous
import functools

import jax
import jax.numpy as jnp
from jax import lax
from jax.experimental import pallas as pl
from jax.experimental.pallas import tpu as pltpu
from jax.experimental.pallas import tpu_sc as plsc

N_NODES = 10000
D = 128
NC = 2
NS = 16
L = 16

GP = 128
C = 160
E_PAD = NS * C * GP
QP = 80
NP = 4
DQ = D // NP
HN = 10240
SROW = HN // DQ
CMAX = C * GP + GP


def _sc_softmax_agg(src2d, dst2d, w2d, featq):
    mesh = plsc.VectorSubcoreMesh(
        core_axis_name="c", subcore_axis_name="s", num_cores=NC, num_subcores=NS
    )

    @functools.partial(
        pl.kernel,
        out_type=jax.ShapeDtypeStruct((NC, NP, HN, DQ), jnp.float32),
        mesh=mesh,
        scratch_types=[
            pltpu.VMEM((QP, GP), jnp.int32),
            pltpu.VMEM((QP, GP), jnp.int32),
            pltpu.VMEM((QP, GP), jnp.float32),
            pltpu.VMEM((SROW, DQ), jnp.float32),
            pltpu.VMEM((SROW, DQ), jnp.float32),
            pltpu.VMEM((CMAX,), jnp.int32),
            pltpu.VMEM((CMAX,), jnp.float32),
            pltpu.VMEM((GP,), jnp.int32),
            pltpu.VMEM((GP,), jnp.int32),
            pltpu.VMEM((GP, DQ), jnp.float32),
            pltpu.VMEM((GP, DQ), jnp.float32),
            pltpu.SemaphoreType.DMA,
            pltpu.VMEM_SHARED((HN, DQ), jnp.float32),
        ],
        compiler_params=pltpu.CompilerParams(
            needs_layout_passes=False, use_tc_tiling_on_sc=False),
    )
    def k(src_h, dst_h, w_h, featq_h, out_h,
          srcp, dstp, wp, sloc, acc, cpack, calpha, gidx, sidx, rows, zbuf,
          gsem, Hs):
        cid = lax.axis_index("c")
        sid = lax.axis_index("s")
        zero16 = jnp.zeros((L,), jnp.float32)
        sgn = jnp.where(cid == 0, 1.0, -1.0).astype(jnp.float32)

        def zz(i, _):
            zbuf[i, pl.ds(0, L)] = zero16
            zbuf[i, pl.ds(L, L)] = zero16
            return 0
        lax.fori_loop(0, GP, zz, 0)

        def zs(i, _):
            sloc[i, pl.ds(0, L)] = zero16
            sloc[i, pl.ds(L, L)] = zero16
            return 0
        lax.fori_loop(0, SROW, zs, 0)

        for q in range(C // QP):
            base = sid * C + q * QP
            pltpu.sync_copy(dst_h.at[pl.ds(base, QP)], dstp)
            pltpu.sync_copy(w_h.at[pl.ds(base, QP)], wp)

            def pa(g, _):
                for j in range(GP // L):
                    sl = pl.ds(j * L, L)
                    dvec = dstp[g, sl]
                    wvec = wp[g, sl]
                    mask = (wvec * sgn) > 0.0
                    val = jnp.exp(jnp.abs(wvec))
                    plsc.addupdate_scatter(
                        sloc, [dvec >> 5, dvec & 31], val, mask=mask)
                return 0
            lax.fori_loop(0, QP, pa, 0)

        pltpu.sync_copy(sloc, Hs.at[pl.ds(sid * SROW, SROW)])
        plsc.subcore_barrier()
        SL = SROW // NS
        for kk in range(NS):
            pltpu.sync_copy(Hs.at[pl.ds(kk * SROW + sid * SL, SL)],
                            acc.at[pl.ds(kk * SL, SL)])

        def rs(r, _):
            for j in range(DQ // L):
                sl = pl.ds(j * L, L)
                t = acc[r, sl]
                for kk in range(1, NS):
                    t = t + acc[kk * SL + r, sl]
                sloc[r, sl] = t
            return 0
        lax.fori_loop(0, SL, rs, 0)
        pltpu.sync_copy(sloc.at[pl.ds(0, SL)],
                        Hs.at[pl.ds(NS * SROW + sid * SL, SL)])
        plsc.subcore_barrier()
        pltpu.sync_copy(Hs.at[pl.ds(NS * SROW, SROW)], sloc)
        plsc.subcore_barrier()

        def zc(i, _):
            cpack[pl.ds(i * L, L)] = jnp.zeros((L,), jnp.int32)
            calpha[pl.ds(i * L, L)] = zero16
            return 0
        lax.fori_loop(0, CMAX // L, zc, 0)

        cnt = jnp.int32(0)
        for q in range(C // QP):
            base = sid * C + q * QP
            pltpu.sync_copy(src_h.at[pl.ds(base, QP)], srcp)
            pltpu.sync_copy(dst_h.at[pl.ds(base, QP)], dstp)
            pltpu.sync_copy(w_h.at[pl.ds(base, QP)], wp)

            def cp(g, cn):
                for j in range(GP // L):
                    sl = pl.ds(j * L, L)
                    svec = srcp[g, sl]
                    dvec = dstp[g, sl]
                    wvec = wp[g, sl]
                    mask = (wvec * sgn) > 0.0
                    val = jnp.exp(jnp.abs(wvec))
                    sv = plsc.load_gather(sloc, [dvec >> 5, dvec & 31])
                    alpha = val / jnp.where(sv > 0.0, sv, 1.0)
                    pk = svec | (dvec << 14)
                    plsc.store_compressed(cpack.at[pl.ds(cn, L)], pk, mask=mask)
                    plsc.store_compressed(calpha.at[pl.ds(cn, L)], alpha, mask=mask)
                    pc = plsc.all_reduce_population_count(mask)
                    cn = cn + pc[0]
                return cn
            cnt = lax.fori_loop(0, QP, cp, cnt)
        ng = (cnt + GP - 1) >> 7

        for p in range(NP):
            for kk in range(HN // NS // GP):
                pltpu.sync_copy(zbuf, Hs.at[pl.ds(sid * (HN // NS) + kk * GP, GP)])
            plsc.subcore_barrier()

            def gb(g, _):
                for j in range(GP // L):
                    sl = pl.ds(j * L, L)
                    pk = cpack[pl.ds(g * GP + j * L, L)]
                    gidx[sl] = (pk & 0x3FFF) * NP + p
                    sidx[sl] = pk >> 14
                pltpu.async_copy(featq_h.at[gidx], rows, gsem).wait()

                def sc(jj, _):
                    avec = calpha[pl.ds(g * GP + jj * L, L)]
                    for rr in range(L):
                        a = avec[rr]
                        r = jj * L + rr
                        rows[r, pl.ds(0, L)] = rows[r, pl.ds(0, L)] * a
                        rows[r, pl.ds(L, L)] = rows[r, pl.ds(L, L)] * a
                    return 0
                lax.fori_loop(0, GP // L, sc, 0)
                pltpu.sync_copy(rows, Hs.at[sidx], add=True)
                return 0
            lax.fori_loop(0, ng, gb, 0)
            plsc.subcore_barrier()

            for kk in range(HN // NS // GP):
                r0 = sid * (HN // NS) + kk * GP
                pltpu.sync_copy(Hs.at[pl.ds(r0, GP)], rows)
                pltpu.sync_copy(rows, out_h.at[cid, p, pl.ds(r0, GP)])

    return k(src2d, dst2d, w2d, featq)


def _tc_combine(feat, hq, w0t, w1t, w2t, b2):
    BM = 1000

    def mk(f_ref, h_ref, w0_ref, w1_ref, w2_ref, b_ref, o_ref):
        a = jnp.dot(f_ref[...], w0_ref[...], preferred_element_type=jnp.float32)
        w1 = w1_ref[...]
        w2 = w2_ref[...]
        for p in range(NP):
            a = a + jnp.dot(h_ref[0, p], w1[p * DQ:(p + 1) * DQ, :],
                            preferred_element_type=jnp.float32)
            a = a + jnp.dot(h_ref[1, p], w2[p * DQ:(p + 1) * DQ, :],
                            preferred_element_type=jnp.float32)
        o_ref[...] = a + b_ref[0]

    return pl.pallas_call(
        mk,
        grid=(N_NODES // BM,),
        in_specs=[
            pl.BlockSpec((BM, D), lambda i: (i, 0)),
            pl.BlockSpec((NC, NP, BM, DQ), lambda i: (0, 0, i, 0)),
            pl.BlockSpec((D, D), lambda i: (0, 0)),
            pl.BlockSpec((D, D), lambda i: (0, 0)),
            pl.BlockSpec((D, D), lambda i: (0, 0)),
            pl.BlockSpec((8, D), lambda i: (0, 0)),
        ],
        out_specs=pl.BlockSpec((BM, D), lambda i: (i, 0)),
        out_shape=jax.ShapeDtypeStruct((N_NODES, D), jnp.float32),
    )(feat, hq, w0t, w1t, w2t, b2)


def kernel(feat, edge_index, edge_weight, W, b_fc, bias,
           coef_self, coef_posi, coef_nega):
    src = edge_index[0]
    dst = edge_index[1]
    pad = E_PAD - src.shape[0]
    src_p = jnp.concatenate([src, jnp.zeros((pad,), jnp.int32)]).reshape(NS * C, GP)
    dst_p = jnp.concatenate([dst, jnp.zeros((pad,), jnp.int32)]).reshape(NS * C, GP)
    w_p = jnp.concatenate(
        [edge_weight, jnp.zeros((pad,), jnp.float32)]).reshape(NS * C, GP)
    featq = feat.reshape(N_NODES * NP, DQ)

    hq = _sc_softmax_agg(src_p, dst_p, w_p, featq)

    w0t = W[:, :D].T * coef_self[0]
    w1t = W[:, D:2 * D].T * coef_posi[0]
    w2t = W[:, 2 * D:].T * coef_nega[0]
    b2 = jnp.broadcast_to((b_fc + bias)[None, :], (8, D))
    return _tc_combine(feat, hq, w0t, w1t, w2t, b2)


# speedup vs baseline: 25.8366x; 1.3702x over previous
import functools

import jax
import jax.numpy as jnp
from jax import lax
from jax.experimental import pallas as pl
from jax.experimental.pallas import tpu as pltpu
from jax.experimental.pallas import tpu_sc as plsc

N_NODES = 10000
D = 128
NC = 2
NS = 16
L = 16

GP = 128
C = 160
E_PAD = NS * C * GP
QP = 40
KG = 2
MG = KG * GP
NP = 4
DQ = D // NP
HN = 10240
SROW = HN // DQ
CMAX = C * GP + MG


def _sc_softmax_agg(src2d, dst2d, w2d, featq):
    mesh = plsc.VectorSubcoreMesh(
        core_axis_name="c", subcore_axis_name="s", num_cores=NC, num_subcores=NS
    )

    @functools.partial(
        pl.kernel,
        out_type=jax.ShapeDtypeStruct((NC, NP, HN, DQ), jnp.float32),
        mesh=mesh,
        scratch_types=[
            pltpu.VMEM((QP, GP), jnp.int32),
            pltpu.VMEM((QP, GP), jnp.int32),
            pltpu.VMEM((QP, GP), jnp.float32),
            pltpu.VMEM((SROW, DQ), jnp.float32),
            pltpu.VMEM((SROW, DQ), jnp.float32),
            pltpu.VMEM((CMAX,), jnp.int32),
            pltpu.VMEM((CMAX,), jnp.float32),
            pltpu.VMEM((KG, GP), jnp.int32),
            pltpu.VMEM((KG, GP), jnp.int32),
            pltpu.VMEM((KG, GP), jnp.int32),
            pltpu.VMEM((KG, GP), jnp.int32),
            pltpu.VMEM((MG, DQ), jnp.float32),
            pltpu.VMEM((MG, DQ), jnp.float32),
            pltpu.VMEM((GP, DQ), jnp.float32),
            pltpu.SemaphoreType.DMA,
            pltpu.SemaphoreType.DMA,
            pltpu.VMEM_SHARED((HN, DQ), jnp.float32),
        ],
        compiler_params=pltpu.CompilerParams(
            needs_layout_passes=False, use_tc_tiling_on_sc=False),
    )
    def k(src_h, dst_h, w_h, featq_h, out_h,
          srcp, dstp, wp, sloc, acc, cpack, calpha,
          gidx0, sidx0, gidx1, sidx1, rows0, rows1, zbuf,
          gsem0, gsem1, Hs):
        cid = lax.axis_index("c")
        sid = lax.axis_index("s")
        zero16 = jnp.zeros((L,), jnp.float32)
        sgn = jnp.where(cid == 0, 1.0, -1.0).astype(jnp.float32)

        def zz(i, _):
            zbuf[i, pl.ds(0, L)] = zero16
            zbuf[i, pl.ds(L, L)] = zero16
            return 0
        lax.fori_loop(0, GP, zz, 0)

        def zs(i, _):
            sloc[i, pl.ds(0, L)] = zero16
            sloc[i, pl.ds(L, L)] = zero16
            return 0
        lax.fori_loop(0, SROW, zs, 0)

        def zc(i, _):
            cpack[pl.ds(i * L, L)] = jnp.zeros((L,), jnp.int32)
            calpha[pl.ds(i * L, L)] = zero16
            return 0
        lax.fori_loop(0, CMAX // L, zc, 0)

        cnt = jnp.int32(0)
        for q in range(C // QP):
            base = sid * C + q * QP
            pltpu.sync_copy(src_h.at[pl.ds(base, QP)], srcp)
            pltpu.sync_copy(dst_h.at[pl.ds(base, QP)], dstp)
            pltpu.sync_copy(w_h.at[pl.ds(base, QP)], wp)

            def pa(g, cn):
                for j in range(GP // L):
                    sl = pl.ds(j * L, L)
                    svec = srcp[g, sl]
                    dvec = dstp[g, sl]
                    wvec = wp[g, sl]
                    mask = (wvec * sgn) > 0.0
                    val = jnp.exp(jnp.abs(wvec))
                    plsc.addupdate_scatter(
                        sloc, [dvec >> 5, dvec & 31], val, mask=mask)
                    pk = svec | (dvec << 14)
                    plsc.store_compressed(cpack.at[pl.ds(cn, L)], pk, mask=mask)
                    plsc.store_compressed(calpha.at[pl.ds(cn, L)], val, mask=mask)
                    pc = plsc.all_reduce_population_count(mask)
                    cn = cn + pc[0]
                return cn
            cnt = lax.fori_loop(0, QP, pa, cnt)

        pltpu.sync_copy(sloc, Hs.at[pl.ds(sid * SROW, SROW)])
        plsc.subcore_barrier()
        SL = SROW // NS
        for kk in range(NS):
            pltpu.sync_copy(Hs.at[pl.ds(kk * SROW + sid * SL, SL)],
                            acc.at[pl.ds(kk * SL, SL)])

        def rs(r, _):
            for j in range(DQ // L):
                sl = pl.ds(j * L, L)
                t = acc[r, sl]
                for kk in range(1, NS):
                    t = t + acc[kk * SL + r, sl]
                sloc[r, sl] = t
            return 0
        lax.fori_loop(0, SL, rs, 0)
        pltpu.sync_copy(sloc.at[pl.ds(0, SL)],
                        Hs.at[pl.ds(NS * SROW + sid * SL, SL)])
        plsc.subcore_barrier()
        pltpu.sync_copy(Hs.at[pl.ds(NS * SROW, SROW)], sloc)
        plsc.subcore_barrier()

        ng128 = (cnt + GP - 1) >> 7

        def cv(g, _):
            for j in range(GP // L):
                sl = pl.ds(g * GP + j * L, L)
                pk = cpack[sl]
                va = calpha[sl]
                dvec = pk >> 14
                sv = plsc.load_gather(sloc, [dvec >> 5, dvec & 31])
                calpha[sl] = va / jnp.where(sv > 0.0, sv, 1.0)
            return 0
        lax.fori_loop(0, ng128, cv, 0)

        ngk = (cnt + MG - 1) >> (MG.bit_length() - 1)

        def unpack(g, gix, six, p):
            for kg in range(KG):
                for j in range(GP // L):
                    sl = pl.ds(j * L, L)
                    pk = cpack[pl.ds(g * MG + kg * GP + j * L, L)]
                    gix[kg, sl] = (pk & 0x3FFF) * NP + p
                    six[kg, sl] = pk >> 14

        def gissue(gix, rbuf, sem):
            def b(kg, _):
                pltpu.async_copy(
                    featq_h.at[gix.at[kg]], rbuf.at[pl.ds(kg * GP, GP)], sem)
                return 0
            lax.fori_loop(0, KG, b, 0)

        def gwait(gix, rbuf, sem):
            def b(kg, _):
                pltpu.make_async_copy(
                    featq_h.at[gix.at[kg]], rbuf.at[pl.ds(kg * GP, GP)], sem
                ).wait()
                return 0
            lax.fori_loop(0, KG, b, 0)

        def sadd(six, rbuf):
            def b(kg, _):
                pltpu.sync_copy(
                    rbuf.at[pl.ds(kg * GP, GP)], Hs.at[six.at[kg]], add=True)
                return 0
            lax.fori_loop(0, KG, b, 0)

        def scale(g, rbuf):
            def sc(t, _):
                avec = calpha[pl.ds(g * MG + t * L, L)]
                for rr in range(L):
                    a = avec[rr]
                    r = t * L + rr
                    rbuf[r, pl.ds(0, L)] = rbuf[r, pl.ds(0, L)] * a
                    rbuf[r, pl.ds(L, L)] = rbuf[r, pl.ds(L, L)] * a
                return 0
            lax.fori_loop(0, MG // L, sc, 0)

        def pass_body(p, _):
            for kk in range(HN // NS // GP):
                pltpu.sync_copy(zbuf, Hs.at[pl.ds(sid * (HN // NS) + kk * GP, GP)])
            plsc.subcore_barrier()

            @pl.when(ngk > 0)
            def _prologue():
                unpack(0, gidx0, sidx0, p)
                gissue(gidx0, rows0, gsem0)

            def pair(i, _):
                g0 = 2 * i
                g1 = 2 * i + 1
                g2 = 2 * i + 2

                @pl.when(g1 < ngk)
                def _issue1():
                    unpack(g1, gidx1, sidx1, p)
                    gissue(gidx1, rows1, gsem1)

                gwait(gidx0, rows0, gsem0)
                scale(g0, rows0)
                sadd(sidx0, rows0)

                @pl.when(g2 < ngk)
                def _issue2():
                    unpack(g2, gidx0, sidx0, p)
                    gissue(gidx0, rows0, gsem0)

                @pl.when(g1 < ngk)
                def _drain1():
                    gwait(gidx1, rows1, gsem1)
                    scale(g1, rows1)
                    sadd(sidx1, rows1)
                return 0

            lax.fori_loop(0, (ngk + 1) >> 1, pair, 0)
            plsc.subcore_barrier()

            r0 = sid * (HN // NS)
            off = 0
            while off < HN // NS:
                sz = min(MG, HN // NS - off)
                pltpu.sync_copy(Hs.at[pl.ds(r0 + off, sz)],
                                rows0.at[pl.ds(0, sz)])
                pltpu.sync_copy(rows0.at[pl.ds(0, sz)],
                                out_h.at[cid, p, pl.ds(r0 + off, sz)])
                off += sz
            return 0

        lax.fori_loop(0, NP, pass_body, 0)

    return k(src2d, dst2d, w2d, featq)


def _tc_combine(feat, hq, w0t, w1t, w2t, b2):
    BM = 1000

    def mk(f_ref, h_ref, w0_ref, w1_ref, w2_ref, b_ref, o_ref):
        a = jnp.dot(f_ref[...], w0_ref[...], preferred_element_type=jnp.float32)
        w1 = w1_ref[...]
        w2 = w2_ref[...]
        for p in range(NP):
            a = a + jnp.dot(h_ref[0, p], w1[p * DQ:(p + 1) * DQ, :],
                            preferred_element_type=jnp.float32)
            a = a + jnp.dot(h_ref[1, p], w2[p * DQ:(p + 1) * DQ, :],
                            preferred_element_type=jnp.float32)
        o_ref[...] = a + b_ref[0]

    return pl.pallas_call(
        mk,
        grid=(N_NODES // BM,),
        in_specs=[
            pl.BlockSpec((BM, D), lambda i: (i, 0)),
            pl.BlockSpec((NC, NP, BM, DQ), lambda i: (0, 0, i, 0)),
            pl.BlockSpec((D, D), lambda i: (0, 0)),
            pl.BlockSpec((D, D), lambda i: (0, 0)),
            pl.BlockSpec((D, D), lambda i: (0, 0)),
            pl.BlockSpec((8, D), lambda i: (0, 0)),
        ],
        out_specs=pl.BlockSpec((BM, D), lambda i: (i, 0)),
        out_shape=jax.ShapeDtypeStruct((N_NODES, D), jnp.float32),
    )(feat, hq, w0t, w1t, w2t, b2)


def kernel(feat, edge_index, edge_weight, W, b_fc, bias,
           coef_self, coef_posi, coef_nega):
    src = edge_index[0]
    dst = edge_index[1]
    pad = E_PAD - src.shape[0]
    src_p = jnp.concatenate([src, jnp.zeros((pad,), jnp.int32)]).reshape(NS * C, GP)
    dst_p = jnp.concatenate([dst, jnp.zeros((pad,), jnp.int32)]).reshape(NS * C, GP)
    w_p = jnp.concatenate(
        [edge_weight, jnp.zeros((pad,), jnp.float32)]).reshape(NS * C, GP)
    featq = feat.reshape(N_NODES * NP, DQ)

    hq = _sc_softmax_agg(src_p, dst_p, w_p, featq)

    w0t = W[:, :D].T * coef_self[0]
    w1t = W[:, D:2 * D].T * coef_posi[0]
    w2t = W[:, 2 * D:].T * coef_nega[0]
    b2 = jnp.broadcast_to((b_fc + bias)[None, :], (8, D))
    return _tc_combine(feat, hq, w0t, w1t, w2t, b2)


# speedup vs baseline: 25.8665x; 1.0012x over previous
import functools

import jax
import jax.numpy as jnp
from jax import lax
from jax.experimental import pallas as pl
from jax.experimental.pallas import tpu as pltpu
from jax.experimental.pallas import tpu_sc as plsc

N_NODES = 10000
D = 128
NC = 2
NS = 16
L = 16

GP = 128
C = 160
E_PAD = NS * C * GP
QP = 40
KG = 2
MG = KG * GP
NP = 4
DQ = D // NP
HN = 10240
SROW = HN // DQ
CMAX = C * GP + MG


def _sc_softmax_agg(src2d, dst2d, w2d, featq):
    mesh = plsc.VectorSubcoreMesh(
        core_axis_name="c", subcore_axis_name="s", num_cores=NC, num_subcores=NS
    )

    @functools.partial(
        pl.kernel,
        out_type=jax.ShapeDtypeStruct((NC, NP, HN, DQ), jnp.float32),
        mesh=mesh,
        scratch_types=[
            pltpu.VMEM((QP, GP), jnp.int32),
            pltpu.VMEM((QP, GP), jnp.int32),
            pltpu.VMEM((QP, GP), jnp.float32),
            pltpu.VMEM((SROW, DQ), jnp.float32),
            pltpu.VMEM((SROW, DQ), jnp.float32),
            pltpu.VMEM((CMAX,), jnp.int32),
            pltpu.VMEM((CMAX,), jnp.float32),
            pltpu.VMEM((KG, GP), jnp.int32),
            pltpu.VMEM((KG, GP), jnp.int32),
            pltpu.VMEM((KG, GP), jnp.int32),
            pltpu.VMEM((KG, GP), jnp.int32),
            pltpu.VMEM((MG, DQ), jnp.float32),
            pltpu.VMEM((MG, DQ), jnp.float32),
            pltpu.VMEM((GP, DQ), jnp.float32),
            pltpu.SemaphoreType.DMA,
            pltpu.SemaphoreType.DMA,
            pltpu.VMEM_SHARED((HN, DQ), jnp.float32),
        ],
        compiler_params=pltpu.CompilerParams(
            needs_layout_passes=False, use_tc_tiling_on_sc=False),
    )
    def k(src_h, dst_h, w_h, featq_h, out_h,
          srcp, dstp, wp, sloc, acc, cpack, calpha,
          gidx0, sidx0, gidx1, sidx1, rows0, rows1, zbuf,
          gsem0, gsem1, Hs):
        cid = lax.axis_index("c")
        sid = lax.axis_index("s")
        zero16 = jnp.zeros((L,), jnp.float32)
        sgn = jnp.where(cid == 0, 1.0, -1.0).astype(jnp.float32)

        def zz(i, _):
            zbuf[i, pl.ds(0, L)] = zero16
            zbuf[i, pl.ds(L, L)] = zero16
            return 0
        lax.fori_loop(0, GP, zz, 0)

        def zs(i, _):
            sloc[i, pl.ds(0, L)] = zero16
            sloc[i, pl.ds(L, L)] = zero16
            return 0
        lax.fori_loop(0, SROW, zs, 0)

        def zc(i, _):
            cpack[pl.ds(i * L, L)] = jnp.zeros((L,), jnp.int32)
            calpha[pl.ds(i * L, L)] = zero16
            return 0
        lax.fori_loop(0, CMAX // L, zc, 0)

        scope_scan = jax.named_scope("edge_scan")
        scope_scan.__enter__()
        cnt = jnp.int32(0)
        for q in range(C // QP):
            base = sid * C + q * QP
            pltpu.sync_copy(src_h.at[pl.ds(base, QP)], srcp)
            pltpu.sync_copy(dst_h.at[pl.ds(base, QP)], dstp)
            pltpu.sync_copy(w_h.at[pl.ds(base, QP)], wp)

            def pa(g, cn):
                for j in range(GP // L):
                    sl = pl.ds(j * L, L)
                    svec = srcp[g, sl]
                    dvec = dstp[g, sl]
                    wvec = wp[g, sl]
                    mask = (wvec * sgn) > 0.0
                    val = jnp.exp(jnp.abs(wvec))
                    plsc.addupdate_scatter(
                        sloc, [dvec >> 5, dvec & 31], val, mask=mask)
                    pk = svec | (dvec << 14)
                    plsc.store_compressed(cpack.at[pl.ds(cn, L)], pk, mask=mask)
                    plsc.store_compressed(calpha.at[pl.ds(cn, L)], val, mask=mask)
                    pc = plsc.all_reduce_population_count(mask)
                    cn = cn + pc[0]
                return cn
            cnt = lax.fori_loop(0, QP, pa, cnt)

        scope_scan.__exit__(None, None, None)
        scope_red = jax.named_scope("s_reduce")
        scope_red.__enter__()
        pltpu.sync_copy(sloc, Hs.at[pl.ds(sid * SROW, SROW)])
        plsc.subcore_barrier()
        SL = SROW // NS
        for kk in range(NS):
            pltpu.sync_copy(Hs.at[pl.ds(kk * SROW + sid * SL, SL)],
                            acc.at[pl.ds(kk * SL, SL)])

        def rs(r, _):
            for j in range(DQ // L):
                sl = pl.ds(j * L, L)
                t = acc[r, sl]
                for kk in range(1, NS):
                    t = t + acc[kk * SL + r, sl]
                sloc[r, sl] = t
            return 0
        lax.fori_loop(0, SL, rs, 0)
        pltpu.sync_copy(sloc.at[pl.ds(0, SL)],
                        Hs.at[pl.ds(NS * SROW + sid * SL, SL)])
        plsc.subcore_barrier()
        pltpu.sync_copy(Hs.at[pl.ds(NS * SROW, SROW)], sloc)
        plsc.subcore_barrier()

        scope_red.__exit__(None, None, None)
        scope_cv = jax.named_scope("convert")
        scope_cv.__enter__()
        ng128 = (cnt + GP - 1) >> 7

        def cv(g, _):
            for j in range(GP // L):
                sl = pl.ds(g * GP + j * L, L)
                pk = cpack[sl]
                va = calpha[sl]
                dvec = pk >> 14
                sv = plsc.load_gather(sloc, [dvec >> 5, dvec & 31])
                calpha[sl] = va / jnp.where(sv > 0.0, sv, 1.0)
            return 0
        lax.fori_loop(0, ng128, cv, 0)

        scope_cv.__exit__(None, None, None)
        ngk = (cnt + MG - 1) >> (MG.bit_length() - 1)

        def unpack(g, gix, six, p):
            for kg in range(KG):
                for j in range(GP // L):
                    sl = pl.ds(j * L, L)
                    pk = cpack[pl.ds(g * MG + kg * GP + j * L, L)]
                    gix[kg, sl] = (pk & 0x3FFF) * NP + p
                    six[kg, sl] = pk >> 14

        def gissue(gix, rbuf, sem):
            def b(kg, _):
                pltpu.async_copy(
                    featq_h.at[gix.at[kg]], rbuf.at[pl.ds(kg * GP, GP)], sem)
                return 0
            lax.fori_loop(0, KG, b, 0)

        def gwait(gix, rbuf, sem):
            def b(kg, _):
                pltpu.make_async_copy(
                    featq_h.at[gix.at[kg]], rbuf.at[pl.ds(kg * GP, GP)], sem
                ).wait()
                return 0
            lax.fori_loop(0, KG, b, 0)

        def sadd(six, rbuf):
            def b(kg, _):
                pltpu.sync_copy(
                    rbuf.at[pl.ds(kg * GP, GP)], Hs.at[six.at[kg]], add=True)
                return 0
            lax.fori_loop(0, KG, b, 0)

        def scale(g, rbuf):
            def sc(t, _):
                avec = calpha[pl.ds(g * MG + t * L, L)]
                for rr in range(L):
                    a = avec[rr]
                    r = t * L + rr
                    rbuf[r, pl.ds(0, L)] = rbuf[r, pl.ds(0, L)] * a
                    rbuf[r, pl.ds(L, L)] = rbuf[r, pl.ds(L, L)] * a
                return 0
            lax.fori_loop(0, MG // L, sc, 0)

        def pass_body(p, _):
            with jax.named_scope("zero"):
                for kk in range(HN // NS // GP):
                    pltpu.sync_copy(zbuf, Hs.at[pl.ds(sid * (HN // NS) + kk * GP, GP)])
                plsc.subcore_barrier()

            @pl.when(ngk > 0)
            def _prologue():
                unpack(0, gidx0, sidx0, p)
                gissue(gidx0, rows0, gsem0)

            def pair(i, _):
                g0 = 2 * i
                g1 = 2 * i + 1
                g2 = 2 * i + 2

                @pl.when(g1 < ngk)
                def _issue1():
                    unpack(g1, gidx1, sidx1, p)
                    gissue(gidx1, rows1, gsem1)

                gwait(gidx0, rows0, gsem0)
                scale(g0, rows0)
                sadd(sidx0, rows0)

                @pl.when(g2 < ngk)
                def _issue2():
                    unpack(g2, gidx0, sidx0, p)
                    gissue(gidx0, rows0, gsem0)

                @pl.when(g1 < ngk)
                def _drain1():
                    gwait(gidx1, rows1, gsem1)
                    scale(g1, rows1)
                    sadd(sidx1, rows1)
                return 0

            with jax.named_scope("agg_loop"):
                lax.fori_loop(0, (ngk + 1) >> 1, pair, 0)
                plsc.subcore_barrier()

            scope_out = jax.named_scope("out_copy")
            scope_out.__enter__()
            r0 = sid * (HN // NS)
            off = 0
            while off < HN // NS:
                sz = min(MG, HN // NS - off)
                pltpu.sync_copy(Hs.at[pl.ds(r0 + off, sz)],
                                rows0.at[pl.ds(0, sz)])
                pltpu.sync_copy(rows0.at[pl.ds(0, sz)],
                                out_h.at[cid, p, pl.ds(r0 + off, sz)])
                off += sz
            scope_out.__exit__(None, None, None)
            return 0

        lax.fori_loop(0, NP, pass_body, 0)

    return k(src2d, dst2d, w2d, featq)


def _tc_combine(feat, hq, w0t, w1t, w2t, b2):
    BM = 1000

    def mk(f_ref, h_ref, w0_ref, w1_ref, w2_ref, b_ref, o_ref):
        a = jnp.dot(f_ref[...], w0_ref[...], preferred_element_type=jnp.float32)
        w1 = w1_ref[...]
        w2 = w2_ref[...]
        for p in range(NP):
            a = a + jnp.dot(h_ref[0, p], w1[p * DQ:(p + 1) * DQ, :],
                            preferred_element_type=jnp.float32)
            a = a + jnp.dot(h_ref[1, p], w2[p * DQ:(p + 1) * DQ, :],
                            preferred_element_type=jnp.float32)
        o_ref[...] = a + b_ref[0]

    return pl.pallas_call(
        mk,
        grid=(N_NODES // BM,),
        in_specs=[
            pl.BlockSpec((BM, D), lambda i: (i, 0)),
            pl.BlockSpec((NC, NP, BM, DQ), lambda i: (0, 0, i, 0)),
            pl.BlockSpec((D, D), lambda i: (0, 0)),
            pl.BlockSpec((D, D), lambda i: (0, 0)),
            pl.BlockSpec((D, D), lambda i: (0, 0)),
            pl.BlockSpec((8, D), lambda i: (0, 0)),
        ],
        out_specs=pl.BlockSpec((BM, D), lambda i: (i, 0)),
        out_shape=jax.ShapeDtypeStruct((N_NODES, D), jnp.float32),
    )(feat, hq, w0t, w1t, w2t, b2)


def kernel(feat, edge_index, edge_weight, W, b_fc, bias,
           coef_self, coef_posi, coef_nega):
    src = edge_index[0]
    dst = edge_index[1]
    pad = E_PAD - src.shape[0]
    src_p = jnp.concatenate([src, jnp.zeros((pad,), jnp.int32)]).reshape(NS * C, GP)
    dst_p = jnp.concatenate([dst, jnp.zeros((pad,), jnp.int32)]).reshape(NS * C, GP)
    w_p = jnp.concatenate(
        [edge_weight, jnp.zeros((pad,), jnp.float32)]).reshape(NS * C, GP)
    featq = feat.reshape(N_NODES * NP, DQ)

    hq = _sc_softmax_agg(src_p, dst_p, w_p, featq)

    w0t = W[:, :D].T * coef_self[0]
    w1t = W[:, D:2 * D].T * coef_posi[0]
    w2t = W[:, 2 * D:].T * coef_nega[0]
    b2 = jnp.broadcast_to((b_fc + bias)[None, :], (8, D))
    return _tc_combine(feat, hq, w0t, w1t, w2t, b2)
